# baseline (speedup 1.0000x reference)
import functools

import jax
import jax.numpy as jnp
from jax import lax
from jax.experimental import pallas as pl
from jax.experimental.pallas import tpu as pltpu
from jax.experimental.pallas import tpu_sc as plsc

_MASK_TOKEN_ID = 32
_MASK_RATIO_TRAIN = 0.15 * 0.8

_NC = 2
_NS = 16
_NW = _NC * _NS

_CHUNK = 32
_NBUF = 3
_LANES = 16


def _prep_body(B, cu_ref, ids_ref, ptf_ref):
    ids = ids_ref[...]
    pos = lax.broadcasted_iota(jnp.int32, ids.shape, 1)
    seg = jnp.zeros(ids.shape, jnp.int32)
    for j in range(1, B):
        seg = seg + jnp.where(pos >= cu_ref[j], 1, 0)
    masked = jnp.where(ids == _MASK_TOKEN_ID, 1.0, 0.0)
    ptf = jnp.zeros(ids.shape, jnp.float32)
    for b in range(B):
        nm = jnp.sum(jnp.where(seg == b, masked, 0.0))
        ln = (cu_ref[b + 1] - cu_ref[b]).astype(jnp.float32)
        scale = (1.0 - _MASK_RATIO_TRAIN) / (1.0 - nm / ln)
        ptf = jnp.where(seg == b, scale, ptf)
    ptf_ref[...] = ptf * (1.0 - masked)


def _gmul_body(nchunk, tok_per_w, ids_hbm, ptf_hbm, tab_hbm, out_hbm,
               idx_v, ptf_v, *rest):
    D = tab_hbm.shape[1]
    bufs = rest[:_NBUF]
    gsems = rest[_NBUF:2 * _NBUF]
    ssems = rest[2 * _NBUF:3 * _NBUF]
    wid = lax.axis_index("c") * _NS + lax.axis_index("s")
    base = wid * tok_per_w
    pltpu.sync_copy(ids_hbm.at[0, pl.ds(base, tok_per_w)], idx_v)
    pltpu.sync_copy(ptf_hbm.at[0, pl.ds(base, tok_per_w)], ptf_v)

    def idxr(g):
        return idx_v.at[pl.ds(g * _CHUNK, _CHUNK)]

    def scale_chunk(g, r):
        def mul_tok(j, carry):
            sp = plsc.load_gather(
                ptf_v, [jnp.full((_LANES,), g * _CHUNK + j, jnp.int32)])
            for k in range(D // _LANES):
                bufs[r][j, pl.ds(k * _LANES, _LANES)] = (
                    bufs[r][j, pl.ds(k * _LANES, _LANES)] * sp)
            return carry
        lax.fori_loop(0, _CHUNK, mul_tok, 0)

    gcp = [None] * nchunk
    scp = [None] * nchunk
    for g in range(_NBUF):
        gcp[g] = pltpu.async_copy(tab_hbm.at[idxr(g)], bufs[g], gsems[g])
    for g in range(nchunk):
        r = g % _NBUF
        gcp[g].wait()
        scale_chunk(g, r)
        scp[g] = pltpu.async_copy(
            bufs[r], out_hbm.at[pl.ds(base + g * _CHUNK, _CHUNK)], ssems[r])
        h = g - 1 + _NBUF
        if g >= 1 and h < nchunk:
            rr = (g - 1) % _NBUF
            scp[g - 1].wait()
            gcp[h] = pltpu.async_copy(tab_hbm.at[idxr(h)], bufs[rr], gsems[rr])
    for g in range(max(0, nchunk - _NBUF), nchunk):
        scp[g].wait()


def kernel(input_ids, cu_seq_lens_q, cu_seq_lens_k, max_length_q, max_length_k, word_embeddings):
    T = input_ids.shape[1]
    V, D = word_embeddings.shape
    B = cu_seq_lens_q.shape[0] - 1
    tok_per_w = T // _NW
    nchunk = tok_per_w // _CHUNK
    assert tok_per_w * _NW == T and nchunk * _CHUNK == tok_per_w

    ptf = pl.pallas_call(
        functools.partial(_prep_body, B),
        out_shape=jax.ShapeDtypeStruct((1, T), jnp.float32),
        in_specs=[
            pl.BlockSpec(memory_space=pltpu.SMEM),
            pl.BlockSpec(memory_space=pltpu.VMEM),
        ],
        out_specs=pl.BlockSpec(memory_space=pltpu.VMEM),
    )(cu_seq_lens_q, input_ids)

    gather = pl.kernel(
        functools.partial(_gmul_body, nchunk, tok_per_w),
        out_type=jax.ShapeDtypeStruct((T, D), jnp.float32),
        compiler_params=pltpu.CompilerParams(needs_layout_passes=False),
        mesh=plsc.VectorSubcoreMesh(core_axis_name="c", subcore_axis_name="s"),
        scratch_types=(
            [pltpu.VMEM((tok_per_w,), jnp.int32),
             pltpu.VMEM((tok_per_w,), jnp.float32)]
            + [pltpu.VMEM((_CHUNK, D), jnp.float32) for _ in range(_NBUF)]
            + [pltpu.SemaphoreType.DMA for _ in range(2 * _NBUF)]
        ),
    )
    out = gather(input_ids, ptf, word_embeddings)
    return out.reshape(1, T, D)


# speedup vs baseline: 1.3986x; 1.3986x over previous
import functools

import jax
import jax.numpy as jnp
from jax import lax
from jax.experimental import pallas as pl
from jax.experimental.pallas import tpu as pltpu
from jax.experimental.pallas import tpu_sc as plsc

_MASK_TOKEN_ID = 32
_MASK_RATIO_TRAIN = 0.15 * 0.8

_NC = 2
_NS = 16
_NW = _NC * _NS

_CHUNK = 16
_NBUF = 6


def _prep_body(cu_ref, ids_ref, tab_ref, tabout_ref, comb_ref):
    V = tab_ref.shape[0]
    B = tabout_ref.shape[0] // V
    ids = ids_ref[...]
    pos = lax.broadcasted_iota(jnp.int32, ids.shape, 1)
    seg = jnp.zeros(ids.shape, jnp.int32)
    for j in range(1, B):
        seg = seg + jnp.where(pos >= cu_ref[j], 1, 0)
    comb_ref[...] = seg * V + ids
    masked = jnp.where(ids == _MASK_TOKEN_ID, 1.0, 0.0)
    tab = tab_ref[...]
    row = lax.broadcasted_iota(jnp.int32, tab.shape, 0)
    tabz = jnp.where(row == _MASK_TOKEN_ID, 0.0, tab)
    for b in range(B):
        nm = jnp.sum(jnp.where(seg == b, masked, 0.0))
        ln = (cu_ref[b + 1] - cu_ref[b]).astype(jnp.float32)
        scale = (1.0 - _MASK_RATIO_TRAIN) / (1.0 - nm / ln)
        tabout_ref[pl.ds(b * V, V), :] = tabz * scale


def _gather_body(nchunk, tok_per_w, comb_hbm, tab_hbm, out_hbm, idx_v, *rest):
    bufs = rest[:_NBUF]
    gsems = rest[_NBUF:2 * _NBUF]
    ssems = rest[2 * _NBUF:3 * _NBUF]
    wid = lax.axis_index("c") * _NS + lax.axis_index("s")
    base = wid * tok_per_w
    pltpu.sync_copy(comb_hbm.at[0, pl.ds(base, tok_per_w)], idx_v)

    def idxr(g):
        return idx_v.at[pl.ds(g * _CHUNK, _CHUNK)]

    gcp = [None] * nchunk
    scp = [None] * nchunk
    for g in range(_NBUF):
        gcp[g] = pltpu.async_copy(tab_hbm.at[idxr(g)], bufs[g], gsems[g])
    for g in range(nchunk):
        r = g % _NBUF
        gcp[g].wait()
        scp[g] = pltpu.async_copy(
            bufs[r], out_hbm.at[pl.ds(base + g * _CHUNK, _CHUNK)], ssems[r])
        h = g - 1 + _NBUF
        if g >= 1 and h < nchunk:
            rr = (g - 1) % _NBUF
            scp[g - 1].wait()
            gcp[h] = pltpu.async_copy(tab_hbm.at[idxr(h)], bufs[rr], gsems[rr])
    for g in range(max(0, nchunk - _NBUF), nchunk):
        scp[g].wait()


def kernel(input_ids, cu_seq_lens_q, cu_seq_lens_k, max_length_q, max_length_k, word_embeddings):
    T = input_ids.shape[1]
    V, D = word_embeddings.shape
    B = cu_seq_lens_q.shape[0] - 1
    tok_per_w = T // _NW
    nchunk = tok_per_w // _CHUNK
    assert tok_per_w * _NW == T and nchunk * _CHUNK == tok_per_w

    scaled, comb = pl.pallas_call(
        _prep_body,
        out_shape=(
            jax.ShapeDtypeStruct((B * V, D), jnp.float32),
            jax.ShapeDtypeStruct((1, T), jnp.int32),
        ),
        in_specs=[
            pl.BlockSpec(memory_space=pltpu.SMEM),
            pl.BlockSpec(memory_space=pltpu.VMEM),
            pl.BlockSpec(memory_space=pltpu.VMEM),
        ],
        out_specs=(
            pl.BlockSpec(memory_space=pltpu.VMEM),
            pl.BlockSpec(memory_space=pltpu.VMEM),
        ),
    )(cu_seq_lens_q, input_ids, word_embeddings)

    gather = pl.kernel(
        functools.partial(_gather_body, nchunk, tok_per_w),
        out_type=jax.ShapeDtypeStruct((T, D), jnp.float32),
        mesh=plsc.VectorSubcoreMesh(core_axis_name="c", subcore_axis_name="s",
                                    num_cores=_NC, num_subcores=_NS),
        scratch_types=(
            [pltpu.VMEM((tok_per_w,), jnp.int32)]
            + [pltpu.VMEM((_CHUNK, D), jnp.float32) for _ in range(_NBUF)]
            + [pltpu.SemaphoreType.DMA for _ in range(2 * _NBUF)]
        ),
    )
    out = gather(comb, scaled)
    return out.reshape(1, T, D)


# speedup vs baseline: 1.6222x; 1.1599x over previous
import functools

import jax
import jax.numpy as jnp
from jax import lax
from jax.experimental import pallas as pl
from jax.experimental.pallas import tpu as pltpu
from jax.experimental.pallas import tpu_sc as plsc

_MASK_TOKEN_ID = 32
_MASK_RATIO_TRAIN = 0.15 * 0.8

_NC = 2
_NS = 16
_NW = _NC * _NS

_CHUNK = 32
_NBUF = 3

_SC_FRAC_NUM, _SC_FRAC_DEN = 1, 2
_BT = 512


def _prep_body(cu_ref, ids_ref, tab_ref, tabout_ref, comb_ref, ptf_ref):
    V = tab_ref.shape[0]
    B = tabout_ref.shape[0] // V
    ids = ids_ref[...]
    pos = lax.broadcasted_iota(jnp.int32, ids.shape, 1)
    seg = jnp.zeros(ids.shape, jnp.int32)
    for j in range(1, B):
        seg = seg + jnp.where(pos >= cu_ref[j], 1, 0)
    comb_ref[...] = seg * V + ids
    masked = jnp.where(ids == _MASK_TOKEN_ID, 1.0, 0.0)
    tab = tab_ref[...]
    row = lax.broadcasted_iota(jnp.int32, tab.shape, 0)
    tabz = jnp.where(row == _MASK_TOKEN_ID, 0.0, tab)
    ptf = jnp.zeros(ids.shape, jnp.float32)
    for b in range(B):
        nm = jnp.sum(jnp.where(seg == b, masked, 0.0))
        ln = (cu_ref[b + 1] - cu_ref[b]).astype(jnp.float32)
        scale = (1.0 - _MASK_RATIO_TRAIN) / (1.0 - nm / ln)
        ptf = jnp.where(seg == b, scale, ptf)
        tabout_ref[pl.ds(b * V, V), :] = tabz * scale
    ptf_ref[...] = ptf * (1.0 - masked)


def _gather_body(nchunk, tok_per_w, comb_hbm, tab_hbm, out_hbm, idx_v, *rest):
    bufs = rest[:_NBUF]
    gsems = rest[_NBUF:2 * _NBUF]
    ssems = rest[2 * _NBUF:3 * _NBUF]
    wid = lax.axis_index("c") * _NS + lax.axis_index("s")
    base = wid * tok_per_w
    pltpu.sync_copy(comb_hbm.at[0, pl.ds(base, tok_per_w)], idx_v)

    def idxr(g):
        return idx_v.at[pl.ds(g * _CHUNK, _CHUNK)]

    gcp = [None] * nchunk
    scp = [None] * nchunk
    for g in range(_NBUF):
        gcp[g] = pltpu.async_copy(tab_hbm.at[idxr(g)], bufs[g], gsems[g])
    for g in range(nchunk):
        r = g % _NBUF
        gcp[g].wait()
        scp[g] = pltpu.async_copy(
            bufs[r], out_hbm.at[pl.ds(base + g * _CHUNK, _CHUNK)], ssems[r])
        h = g - 1 + _NBUF
        if g >= 1 and h < nchunk:
            rr = (g - 1) % _NBUF
            scp[g - 1].wait()
            gcp[h] = pltpu.async_copy(tab_hbm.at[idxr(h)], bufs[rr], gsems[rr])
    for g in range(max(0, nchunk - _NBUF), nchunk):
        scp[g].wait()


def _fill_body(sc_out_ref, ids_ref, ptf_ref, tab_ref, out_ref):
    del sc_out_ref
    V, D = tab_ref.shape
    BT = ids_ref.shape[1]
    ids = ids_ref[...].reshape(BT, 1)
    ptf = ptf_ref[...].reshape(BT, 1)
    vocab = lax.broadcasted_iota(jnp.int32, (BT, V), 1)
    onehot = jnp.where(ids == vocab, ptf, 0.0)
    tab = tab_ref[...]
    row = lax.broadcasted_iota(jnp.int32, tab.shape, 0)
    tabz = jnp.where(row == _MASK_TOKEN_ID, 0.0, tab)
    out_ref[...] = jnp.dot(onehot, tabz,
                           preferred_element_type=jnp.float32)


def kernel(input_ids, cu_seq_lens_q, cu_seq_lens_k, max_length_q, max_length_k, word_embeddings):
    T = input_ids.shape[1]
    V, D = word_embeddings.shape
    B = cu_seq_lens_q.shape[0] - 1
    T1 = (T * _SC_FRAC_NUM // _SC_FRAC_DEN)
    tok_per_w = T1 // _NW
    nchunk = tok_per_w // _CHUNK
    nb = (T - T1) // _BT
    assert tok_per_w * _NW == T1 and nchunk * _CHUNK == tok_per_w
    assert T1 % _BT == 0 and nb * _BT == T - T1

    scaled, comb, ptf = pl.pallas_call(
        _prep_body,
        out_shape=(
            jax.ShapeDtypeStruct((B * V, D), jnp.float32),
            jax.ShapeDtypeStruct((1, T), jnp.int32),
            jax.ShapeDtypeStruct((1, T), jnp.float32),
        ),
        in_specs=[
            pl.BlockSpec(memory_space=pltpu.SMEM),
            pl.BlockSpec(memory_space=pltpu.VMEM),
            pl.BlockSpec(memory_space=pltpu.VMEM),
        ],
        out_specs=(
            pl.BlockSpec(memory_space=pltpu.VMEM),
            pl.BlockSpec(memory_space=pltpu.VMEM),
            pl.BlockSpec(memory_space=pltpu.VMEM),
        ),
    )(cu_seq_lens_q, input_ids, word_embeddings)

    gather = pl.kernel(
        functools.partial(_gather_body, nchunk, tok_per_w),
        out_type=jax.ShapeDtypeStruct((T, D), jnp.float32),
        mesh=plsc.VectorSubcoreMesh(core_axis_name="c", subcore_axis_name="s",
                                    num_cores=_NC, num_subcores=_NS),
        scratch_types=(
            [pltpu.VMEM((tok_per_w,), jnp.int32)]
            + [pltpu.VMEM((_CHUNK, D), jnp.float32) for _ in range(_NBUF)]
            + [pltpu.SemaphoreType.DMA for _ in range(2 * _NBUF)]
        ),
    )
    sc_out = gather(comb, scaled)

    blk0 = T1 // _BT
    out = pl.pallas_call(
        _fill_body,
        grid=(nb,),
        out_shape=jax.ShapeDtypeStruct((T, D), jnp.float32),
        in_specs=[
            pl.BlockSpec(memory_space=pl.ANY),
            pl.BlockSpec((1, _BT), lambda i: (0, blk0 + i)),
            pl.BlockSpec((1, _BT), lambda i: (0, blk0 + i)),
            pl.BlockSpec((V, D), lambda i: (0, 0)),
        ],
        out_specs=pl.BlockSpec((_BT, D), lambda i: (blk0 + i, 0)),
        input_output_aliases={0: 0},
    )(sc_out, input_ids, ptf, word_embeddings)
    return out.reshape(1, T, D)
